# baseline (speedup 1.0000x reference)
import jax
from functools import partial
import jax.numpy as jnp
from jax.experimental import pallas as pl
from jax.experimental.pallas import tpu as pltpu

_INV_T = 5.0
_QB = 1024
_BB = 4


def _norm_rows(x, axis):
    return x * jnp.minimum(
        jax.lax.rsqrt(jnp.sum(x * x, axis=axis, keepdims=True)), 1e12)


def _corr_body(d1_ref, d2_ref, qn_ref, pos_ref):
    xn = _norm_rows(d1_ref[...], 1)
    yn = _norm_rows(d2_ref[...], 1)
    qn_ref[...] = xn
    m = xn.shape[0]
    n = yn.shape[0]
    sim = jax.lax.dot_general(xn, yn, (((1,), (1,)), ((), ())),
                              preferred_element_type=jnp.float32)
    ri = jax.lax.broadcasted_iota(jnp.int32, (m, n), 0)
    ci = jax.lax.broadcasted_iota(jnp.int32, (m, n), 1)
    rb = jax.lax.shift_right_logical(ri * 5351, 20)
    cb = jax.lax.shift_right_logical(ci * 5351, 20)
    sim = jnp.where(rb == cb, sim, -2.0)
    pos_ref[...] = jnp.max(sim, axis=1, keepdims=True) * _INV_T


def _neg_body(nq, qn_ref, pos_ref, queue_ref, neg_ref, loss_ref, acc_ref):
    j = pl.program_id(0)
    qbn = _norm_rows(queue_ref[...], 1)
    neg = jax.lax.dot_general(qn_ref[...], qbn, (((1,), (1,)), ((), ())),
                              preferred_element_type=jnp.float32) * _INV_T
    neg_ref[...] = neg
    ex = jnp.exp(neg)
    lp = ex[:, 0:128]
    for k in range(1, _QB // 128):
        lp = lp + ex[:, k * 128:(k + 1) * 128]

    @pl.when(j == 0)
    def _():
        acc_ref[...] = lp

    @pl.when(j > 0)
    def _():
        acc_ref[...] += lp

    @pl.when(j == nq - 1)
    def _():
        pos = pos_ref[...]
        tot = jnp.sum(acc_ref[...], axis=1, keepdims=True) + jnp.exp(pos)
        per_row = jnp.log(tot) - pos
        loss_ref[...] = (jnp.sum(per_row, axis=0, keepdims=True)
                         * (1.0 / pos.shape[0]))


@jax.jit
def _impl(d1, d2, queue):
    B, H, W, D = d1.shape
    N = H * W
    R = B * N
    Q = queue.shape[0]

    d1r = d1.reshape(R, D)
    d2r = d2.reshape(R, D)
    rb = _BB * N

    qn, pos = pl.pallas_call(
        _corr_body,
        grid=(B // _BB,),
        in_specs=[pl.BlockSpec((rb, D), lambda b: (b, 0)),
                  pl.BlockSpec((rb, D), lambda b: (b, 0))],
        out_specs=[pl.BlockSpec((rb, D), lambda b: (b, 0)),
                   pl.BlockSpec((rb, 1), lambda b: (b, 0))],
        out_shape=[jax.ShapeDtypeStruct((R, D), jnp.float32),
                   jax.ShapeDtypeStruct((R, 1), jnp.float32)],
        compiler_params=pltpu.CompilerParams(
            dimension_semantics=("parallel",)),
        name="corr_pos",
    )(d1r, d2r)
    nq = Q // _QB
    neg, loss = pl.pallas_call(
        partial(_neg_body, nq),
        grid=(nq,),
        in_specs=[pl.BlockSpec((R, D), lambda j: (0, 0)),
                  pl.BlockSpec((R, 1), lambda j: (0, 0)),
                  pl.BlockSpec((_QB, D), lambda j: (j, 0))],
        out_specs=[pl.BlockSpec((R, _QB), lambda j: (0, j)),
                   pl.BlockSpec((1, 1), lambda j: (0, 0))],
        out_shape=[jax.ShapeDtypeStruct((R, Q), jnp.float32),
                   jax.ShapeDtypeStruct((1, 1), jnp.float32)],
        scratch_shapes=[pltpu.VMEM((R, 128), jnp.float32)],
        compiler_params=pltpu.CompilerParams(
            dimension_semantics=("arbitrary",),
            vmem_limit_bytes=50 * 1024 * 1024),
        name="neg_queue",
    )(qn, pos, queue)

    return loss[0, 0], pos.reshape(B, N), neg


def kernel(dense_features_1, dense_features_2, backbone_features_1,
           backbone_features_2, queue):
    del backbone_features_1, backbone_features_2
    return _impl(dense_features_1, dense_features_2, queue)


# speedup vs baseline: 1.0225x; 1.0225x over previous
import jax
from functools import partial
import jax.numpy as jnp
from jax.experimental import pallas as pl
from jax.experimental.pallas import tpu as pltpu

_INV_T = 5.0
_QB = 1024
_CB = 784


def _norm_rows(x, axis):
    return x * jnp.minimum(
        jax.lax.rsqrt(jnp.sum(x * x, axis=axis, keepdims=True)), 1e12)


def _body(nq, d1_ref, d2_ref, queue_ref, neg_ref, pos_ref, loss_ref,
          qn_ref, acc_ref):
    j = pl.program_id(0)

    @pl.when(j == 0)
    def _():
        xn = _norm_rows(d1_ref[...], 1)
        yn = _norm_rows(d2_ref[...], 1)
        qn_ref[...] = xn
        ri = jax.lax.broadcasted_iota(jnp.int32, (_CB, _CB), 0)
        ci = jax.lax.broadcasted_iota(jnp.int32, (_CB, _CB), 1)
        same = (jax.lax.shift_right_logical(ri * 5351, 20)
                == jax.lax.shift_right_logical(ci * 5351, 20))
        for c in range(d1_ref.shape[0] // _CB):
            lo, hi = c * _CB, (c + 1) * _CB
            sim = jax.lax.dot_general(xn[lo:hi], yn[lo:hi],
                                      (((1,), (1,)), ((), ())),
                                      preferred_element_type=jnp.float32)
            sim = jnp.where(same, sim, -2.0)
            pos_ref[lo:hi, :] = jnp.max(sim, axis=1, keepdims=True) * _INV_T

    qbn = _norm_rows(queue_ref[...], 1)
    neg = jax.lax.dot_general(qn_ref[...], qbn, (((1,), (1,)), ((), ())),
                              preferred_element_type=jnp.float32) * _INV_T
    neg_ref[...] = neg
    ex = jnp.exp(neg)
    lp = ex[:, 0:128]
    for k in range(1, _QB // 128):
        lp = lp + ex[:, k * 128:(k + 1) * 128]

    @pl.when(j == 0)
    def _():
        acc_ref[...] = lp

    @pl.when(j > 0)
    def _():
        acc_ref[...] += lp

    @pl.when(j == nq - 1)
    def _():
        pos = pos_ref[...]
        tot = jnp.sum(acc_ref[...], axis=1, keepdims=True) + jnp.exp(pos)
        per_row = jnp.log(tot) - pos
        loss_ref[...] = (jnp.sum(per_row, axis=0, keepdims=True)
                         * (1.0 / pos.shape[0]))


@jax.jit
def _impl(d1, d2, queue):
    B, H, W, D = d1.shape
    N = H * W
    R = B * N
    Q = queue.shape[0]
    nq = Q // _QB

    d1r = d1.reshape(R, D)
    d2r = d2.reshape(R, D)

    neg, pos, loss = pl.pallas_call(
        partial(_body, nq),
        grid=(nq,),
        in_specs=[pl.BlockSpec((R, D), lambda j: (0, 0)),
                  pl.BlockSpec((R, D), lambda j: (0, 0)),
                  pl.BlockSpec((_QB, D), lambda j: (j, 0))],
        out_specs=[pl.BlockSpec((R, _QB), lambda j: (0, j)),
                   pl.BlockSpec((R, 1), lambda j: (0, 0)),
                   pl.BlockSpec((1, 1), lambda j: (0, 0))],
        out_shape=[jax.ShapeDtypeStruct((R, Q), jnp.float32),
                   jax.ShapeDtypeStruct((R, 1), jnp.float32),
                   jax.ShapeDtypeStruct((1, 1), jnp.float32)],
        scratch_shapes=[pltpu.VMEM((R, D), jnp.float32),
                        pltpu.VMEM((R, 128), jnp.float32)],
        compiler_params=pltpu.CompilerParams(
            dimension_semantics=("arbitrary",),
            vmem_limit_bytes=50 * 1024 * 1024),
        name="neg_queue",
    )(d1r, d2r, queue)

    return loss[0, 0], pos.reshape(B, N), neg


def kernel(dense_features_1, dense_features_2, backbone_features_1,
           backbone_features_2, queue):
    del backbone_features_1, backbone_features_2
    return _impl(dense_features_1, dense_features_2, queue)


# speedup vs baseline: 1.0323x; 1.0096x over previous
import jax
from functools import partial
import jax.numpy as jnp
from jax.experimental import pallas as pl
from jax.experimental.pallas import tpu as pltpu

_INV_T = 5.0
_QB = 1024
_CB = 784


def _norm_rows(x, axis):
    return x * jnp.minimum(
        jax.lax.rsqrt(jnp.sum(x * x, axis=axis, keepdims=True)), 1e12)


def _body(nq, d1_ref, d2_ref, queue_ref, neg_ref, pos_ref, loss_ref,
          qn_ref, acc_ref):
    j = pl.program_id(0)

    @pl.when(j == 0)
    def _():
        R, D = qn_ref.shape
        nb = R // 196
        x = d1_ref[...].reshape(196, nb, D).swapaxes(0, 1).reshape(R, D)
        y = d2_ref[...].reshape(196, nb, D).swapaxes(0, 1).reshape(R, D)
        xn = _norm_rows(x, 1)
        yn = _norm_rows(y, 1)
        qn_ref[...] = xn
        ri = jax.lax.broadcasted_iota(jnp.int32, (_CB, _CB), 0)
        ci = jax.lax.broadcasted_iota(jnp.int32, (_CB, _CB), 1)
        same = (jax.lax.shift_right_logical(ri * 5351, 20)
                == jax.lax.shift_right_logical(ci * 5351, 20))
        for c in range(d1_ref.shape[0] // _CB):
            lo, hi = c * _CB, (c + 1) * _CB
            sim = jax.lax.dot_general(xn[lo:hi], yn[lo:hi],
                                      (((1,), (1,)), ((), ())),
                                      preferred_element_type=jnp.float32)
            sim = jnp.where(same, sim, -2.0)
            pos_ref[lo:hi, :] = jnp.max(sim, axis=1, keepdims=True) * _INV_T

    qbn = _norm_rows(queue_ref[...], 1)
    neg = jax.lax.dot_general(qn_ref[...], qbn, (((1,), (1,)), ((), ())),
                              preferred_element_type=jnp.float32) * _INV_T
    neg_ref[...] = neg
    ex = jnp.exp(neg)
    lp = ex[:, 0:128]
    for k in range(1, _QB // 128):
        lp = lp + ex[:, k * 128:(k + 1) * 128]

    @pl.when(j == 0)
    def _():
        acc_ref[...] = lp

    @pl.when(j > 0)
    def _():
        acc_ref[...] += lp

    @pl.when(j == nq - 1)
    def _():
        pos = pos_ref[...]
        tot = jnp.sum(acc_ref[...], axis=1, keepdims=True) + jnp.exp(pos)
        per_row = jnp.log(tot) - pos
        loss_ref[...] = (jnp.sum(per_row, axis=0, keepdims=True)
                         * (1.0 / pos.shape[0]))


@jax.jit
def _impl(d1, d2, queue):
    B, H, W, D = d1.shape
    N = H * W
    R = B * N
    Q = queue.shape[0]
    nq = Q // _QB

    d1r = d1.reshape(B, N, D).transpose(1, 0, 2).reshape(R, D)
    d2r = d2.reshape(B, N, D).transpose(1, 0, 2).reshape(R, D)

    neg, pos, loss = pl.pallas_call(
        partial(_body, nq),
        grid=(nq,),
        in_specs=[pl.BlockSpec((R, D), lambda j: (0, 0)),
                  pl.BlockSpec((R, D), lambda j: (0, 0)),
                  pl.BlockSpec((_QB, D), lambda j: (j, 0))],
        out_specs=[pl.BlockSpec((R, _QB), lambda j: (0, j)),
                   pl.BlockSpec((R, 1), lambda j: (0, 0)),
                   pl.BlockSpec((1, 1), lambda j: (0, 0))],
        out_shape=[jax.ShapeDtypeStruct((R, Q), jnp.float32),
                   jax.ShapeDtypeStruct((R, 1), jnp.float32),
                   jax.ShapeDtypeStruct((1, 1), jnp.float32)],
        scratch_shapes=[pltpu.VMEM((R, D), jnp.float32),
                        pltpu.VMEM((R, 128), jnp.float32)],
        compiler_params=pltpu.CompilerParams(
            dimension_semantics=("arbitrary",),
            vmem_limit_bytes=50 * 1024 * 1024),
        name="neg_queue",
    )(d1r, d2r, queue)

    return loss[0, 0], pos.reshape(B, N), neg


def kernel(dense_features_1, dense_features_2, backbone_features_1,
           backbone_features_2, queue):
    del backbone_features_1, backbone_features_2
    return _impl(dense_features_1, dense_features_2, queue)
